# baseline (speedup 1.0000x reference)
import functools

import jax
import jax.numpy as jnp
from jax import lax
from jax.experimental import pallas as pl
from jax.experimental.pallas import tpu as pltpu, tpu_sc as plsc

_MU = 5.0
_D = 256
_C = 64
_NW = 32


def _tec_body(xT_hbm, xS_hbm, p_hbm, n_hbm, yN_hbm, out_hbm,
              pv, nv, y_v, tIdx_v, sIdx_v, wA_v, wB_v,
              rowsT_v, rowsS_v, tr_v, acc_v, semsT, semsS,
              *, num_p, num_n):
    wid = lax.axis_index("s") * 2 + lax.axis_index("c")
    hpw = num_p // _NW
    ppw = 2 * hpw
    nchunks = ppw // _C
    lane = lax.iota(jnp.int32, 16)
    zero16 = jnp.zeros((16,), jnp.float32)

    pltpu.sync_copy(p_hbm.at[pl.ds(wid * hpw, hpw)], pv)
    pltpu.sync_copy(n_hbm.at[pl.ds(wid * hpw, hpw)], nv)
    pltpu.sync_copy(yN_hbm.at[pl.ds(wid * hpw, hpw)], y_v)

    col0 = jnp.zeros((16,), jnp.int32)
    col1 = jnp.ones((16,), jnp.int32)
    wa_p = jnp.full((16,), 1.0 / num_p, jnp.float32)
    inv_n = 1.0 / num_n

    @pl.loop(0, hpw // 16)
    def stage_loop(i):
        rows = i * 16 + lane
        off = i * 16
        tIdx_v[pl.ds(off, 16)] = plsc.load_gather(pv, [rows, col0])
        sIdx_v[pl.ds(off, 16)] = plsc.load_gather(pv, [rows, col1])
        wA_v[pl.ds(off, 16)] = wa_p
        wB_v[pl.ds(off, 16)] = zero16
        off2 = hpw + off
        tIdx_v[pl.ds(off2, 16)] = plsc.load_gather(nv, [rows, col0])
        sIdx_v[pl.ds(off2, 16)] = plsc.load_gather(nv, [rows, col1])
        y = y_v[pl.ds(off, 16)]
        wA_v[pl.ds(off2, 16)] = y * inv_n
        wB_v[pl.ds(off2, 16)] = (1.0 - y) * inv_n

    def start(c, b):
        pltpu.async_copy(xT_hbm.at[tIdx_v.at[pl.ds(c * _C, _C)]],
                         rowsT_v.at[b], semsT.at[b])
        pltpu.async_copy(xS_hbm.at[sIdx_v.at[pl.ds(c * _C, _C)]],
                         rowsS_v.at[b], semsS.at[b])

    def wait(b):
        pltpu.make_async_copy(xT_hbm.at[pl.ds(0, _C)], rowsT_v.at[b],
                              semsT.at[b]).wait()
        pltpu.make_async_copy(xS_hbm.at[pl.ds(0, _C)], rowsS_v.at[b],
                              semsS.at[b]).wait()

    lane17 = lane * 17

    def compute(c, b, acc_total):
        for g in range(_C // 16):
            @pl.loop(0, 16)
            def p_loop(p):
                row = g * 16 + p
                acc = zero16
                for k in range(_D // 32):
                    a = rowsT_v[b, row, pl.ds(k * 32, 32)]
                    bb = rowsS_v[b, row, pl.ds(k * 32, 32)]
                    d = a - bb
                    dlo, dhi = plsc.unpack(d, format=plsc.PackFormat.INTERLEAVED)
                    acc = acc + dlo * dlo + dhi * dhi
                tr_v[pl.ds(p * 17, 16)] = acc
            d2 = plsc.load_gather(tr_v, [lane17])
            for k in range(1, 16):
                d2 = d2 + plsc.load_gather(tr_v, [lane17 + k])
            wA = wA_v[pl.ds(c * _C + g * 16, 16)]
            wB = wB_v[pl.ds(c * _C + g * 16, 16)]
            acc_total = acc_total + wA * d2 + wB * jnp.maximum(_MU - d2, 0.0)
        return acc_total

    start(0, 0)

    @pl.loop(0, nchunks // 2, init_carry=zero16)
    def chunk_loop(h, acc_total):
        for b in range(2):
            c = 2 * h + b

            @pl.when(c + 1 < nchunks)
            def _():
                start(c + 1, 1 - b)

            wait(b)
            acc_total = compute(c, b, acc_total)
        return acc_total

    acc_v[...] = chunk_loop
    pltpu.sync_copy(acc_v, out_hbm.at[wid])


def kernel(xS, xT, p_, n_):
    numP = p_.shape[0]
    numN = n_.shape[0]
    assert numP % (_NW * 16) == 0 and numN % (_NW * 16) == 0
    assert (numP + numN) % (_NW * 2 * _C) == 0
    hpw = numP // _NW

    yN = 0.2 * jax.random.uniform(jax.random.key(42), (numN,), dtype=jnp.float32)

    mesh = plsc.VectorSubcoreMesh(core_axis_name="c", subcore_axis_name="s")
    run = pl.kernel(
        functools.partial(_tec_body, num_p=numP, num_n=numN),
        out_type=jax.ShapeDtypeStruct((_NW, 16), jnp.float32),
        mesh=mesh,
        compiler_params=pltpu.CompilerParams(use_tc_tiling_on_sc=False,
                                             needs_layout_passes=False),
        scratch_types=[
            pltpu.VMEM((hpw, 2), jnp.int32),
            pltpu.VMEM((hpw, 2), jnp.int32),
            pltpu.VMEM((hpw,), jnp.float32),
            pltpu.VMEM((2 * hpw,), jnp.int32),
            pltpu.VMEM((2 * hpw,), jnp.int32),
            pltpu.VMEM((2 * hpw,), jnp.float32),
            pltpu.VMEM((2 * hpw,), jnp.float32),
            pltpu.VMEM((2, _C, _D), jnp.bfloat16),
            pltpu.VMEM((2, _C, _D), jnp.bfloat16),
            pltpu.VMEM((16 * 17,), jnp.float32),
            pltpu.VMEM((16,), jnp.float32),
            pltpu.SemaphoreType.DMA((2,)),
            pltpu.SemaphoreType.DMA((2,)),
        ],
    )
    partials = run(xT.astype(jnp.bfloat16), xS.astype(jnp.bfloat16),
                   p_.astype(jnp.int32), n_.astype(jnp.int32), yN)
    return jnp.sum(partials, dtype=jnp.float32).reshape((1,))


# speedup vs baseline: 1.6837x; 1.6837x over previous
import functools

import jax
import jax.numpy as jnp
from jax import lax
from jax.experimental import pallas as pl
from jax.experimental.pallas import tpu as pltpu, tpu_sc as plsc

_MU = 5.0
_D = 256
_C = 64
_NW = 32


def _tec_body(xT_hbm, xS_hbm, tIdx_hbm, sIdx_hbm, wA_hbm, wB_hbm, out_hbm,
              tIdx_v, sIdx_v, wA_v, wB_v, rowsT_v, rowsS_v, tr_v, acc_v,
              semsT, semsS, *, ppw):
    wid = lax.axis_index("s") * 2 + lax.axis_index("c")
    base = wid * ppw
    nchunks = ppw // _C
    lane = lax.iota(jnp.int32, 16)
    zero16 = jnp.zeros((16,), jnp.float32)

    pltpu.sync_copy(tIdx_hbm.at[pl.ds(base, ppw)], tIdx_v)
    pltpu.sync_copy(sIdx_hbm.at[pl.ds(base, ppw)], sIdx_v)
    pltpu.sync_copy(wA_hbm.at[pl.ds(base, ppw)], wA_v)
    pltpu.sync_copy(wB_hbm.at[pl.ds(base, ppw)], wB_v)

    def start(c, b):
        pltpu.async_copy(xT_hbm.at[tIdx_v.at[pl.ds(c * _C, _C)]],
                         rowsT_v.at[b], semsT.at[b])
        pltpu.async_copy(xS_hbm.at[sIdx_v.at[pl.ds(c * _C, _C)]],
                         rowsS_v.at[b], semsS.at[b])

    def wait(b):
        pltpu.make_async_copy(xT_hbm.at[pl.ds(0, _C)], rowsT_v.at[b],
                              semsT.at[b]).wait()
        pltpu.make_async_copy(xS_hbm.at[pl.ds(0, _C)], rowsS_v.at[b],
                              semsS.at[b]).wait()

    lane17 = lane * 17

    def compute(c, b, acc_total):
        for g in range(_C // 16):
            @pl.loop(0, 16)
            def p_loop(p):
                row = g * 16 + p
                acc = zero16
                for k in range(_D // 32):
                    a = rowsT_v[b, row, pl.ds(k * 32, 32)]
                    bb = rowsS_v[b, row, pl.ds(k * 32, 32)]
                    d = a - bb
                    dlo, dhi = plsc.unpack(d, format=plsc.PackFormat.INTERLEAVED)
                    acc = acc + dlo * dlo + dhi * dhi
                tr_v[pl.ds(p * 17, 16)] = acc
            d2 = plsc.load_gather(tr_v, [lane17])
            for k in range(1, 16):
                d2 = d2 + plsc.load_gather(tr_v, [lane17 + k])
            wA = wA_v[pl.ds(c * _C + g * 16, 16)]
            wB = wB_v[pl.ds(c * _C + g * 16, 16)]
            acc_total = acc_total + wA * d2 + wB * jnp.maximum(_MU - d2, 0.0)
        return acc_total

    start(0, 0)

    @pl.loop(0, nchunks // 2, init_carry=zero16)
    def chunk_loop(h, acc_total):
        for b in range(2):
            c = 2 * h + b

            @pl.when(c + 1 < nchunks)
            def _():
                start(c + 1, 1 - b)

            wait(b)
            acc_total = compute(c, b, acc_total)
        return acc_total

    acc_v[...] = chunk_loop
    pltpu.sync_copy(acc_v, out_hbm.at[wid])


def kernel(xS, xT, p_, n_):
    numP = p_.shape[0]
    numN = n_.shape[0]
    total = numP + numN
    assert total % (_NW * 2 * _C) == 0
    ppw = total // _NW

    yN = 0.2 * jax.random.uniform(jax.random.key(42), (numN,), dtype=jnp.float32)
    tIdx = jnp.concatenate([p_[:, 0], n_[:, 0]]).astype(jnp.int32)
    sIdx = jnp.concatenate([p_[:, 1], n_[:, 1]]).astype(jnp.int32)
    wA = jnp.concatenate([jnp.full((numP,), 1.0 / numP, jnp.float32),
                          yN / numN])
    wB = jnp.concatenate([jnp.zeros((numP,), jnp.float32),
                          (1.0 - yN) / numN])

    mesh = plsc.VectorSubcoreMesh(core_axis_name="c", subcore_axis_name="s")
    run = pl.kernel(
        functools.partial(_tec_body, ppw=ppw),
        out_type=jax.ShapeDtypeStruct((_NW, 16), jnp.float32),
        mesh=mesh,
        compiler_params=pltpu.CompilerParams(use_tc_tiling_on_sc=False,
                                             needs_layout_passes=False),
        scratch_types=[
            pltpu.VMEM((ppw,), jnp.int32),
            pltpu.VMEM((ppw,), jnp.int32),
            pltpu.VMEM((ppw,), jnp.float32),
            pltpu.VMEM((ppw,), jnp.float32),
            pltpu.VMEM((2, _C, _D), jnp.bfloat16),
            pltpu.VMEM((2, _C, _D), jnp.bfloat16),
            pltpu.VMEM((16 * 17,), jnp.float32),
            pltpu.VMEM((16,), jnp.float32),
            pltpu.SemaphoreType.DMA((2,)),
            pltpu.SemaphoreType.DMA((2,)),
        ],
    )
    partials = run(xT.astype(jnp.bfloat16), xS.astype(jnp.bfloat16),
                   tIdx, sIdx, wA, wB)
    return jnp.sum(partials, dtype=jnp.float32).reshape((1,))


# speedup vs baseline: 1.6895x; 1.0034x over previous
import functools

import jax
import jax.numpy as jnp
from jax import lax
from jax.experimental import pallas as pl
from jax.experimental.pallas import tpu as pltpu, tpu_sc as plsc

_MU = 5.0
_D = 256
_C = 64
_NW = 32


def _tec_body(xT_hbm, xS_hbm, idx_hbm, w_hbm, out_hbm,
              tIdx_v, sIdx_v, wA_v, wB_v, rowsT_v, rowsS_v, tr_v, acc_v,
              semsT, semsS, *, ppw, total):
    wid = lax.axis_index("s") * 2 + lax.axis_index("c")
    base = wid * ppw
    nchunks = ppw // _C
    lane = lax.iota(jnp.int32, 16)
    zero16 = jnp.zeros((16,), jnp.float32)

    pltpu.sync_copy(idx_hbm.at[pl.ds(base, ppw)], tIdx_v)
    pltpu.sync_copy(idx_hbm.at[pl.ds(total + base, ppw)], sIdx_v)
    pltpu.sync_copy(w_hbm.at[pl.ds(base, ppw)], wA_v)
    pltpu.sync_copy(w_hbm.at[pl.ds(total + base, ppw)], wB_v)

    def start(c, b):
        pltpu.async_copy(xT_hbm.at[tIdx_v.at[pl.ds(c * _C, _C)]],
                         rowsT_v.at[b], semsT.at[b])
        pltpu.async_copy(xS_hbm.at[sIdx_v.at[pl.ds(c * _C, _C)]],
                         rowsS_v.at[b], semsS.at[b])

    def wait(b):
        pltpu.make_async_copy(xT_hbm.at[pl.ds(0, _C)], rowsT_v.at[b],
                              semsT.at[b]).wait()
        pltpu.make_async_copy(xS_hbm.at[pl.ds(0, _C)], rowsS_v.at[b],
                              semsS.at[b]).wait()

    lane17 = lane * 17

    def compute(c, b, acc_total):
        for g in range(_C // 16):
            @pl.loop(0, 16)
            def p_loop(p):
                row = g * 16 + p
                acc = zero16
                for k in range(_D // 32):
                    a = rowsT_v[b, row, pl.ds(k * 32, 32)]
                    bb = rowsS_v[b, row, pl.ds(k * 32, 32)]
                    d = a - bb
                    dlo, dhi = plsc.unpack(d, format=plsc.PackFormat.INTERLEAVED)
                    acc = acc + dlo * dlo + dhi * dhi
                tr_v[pl.ds(p * 17, 16)] = acc
            d2 = plsc.load_gather(tr_v, [lane17])
            for k in range(1, 16):
                d2 = d2 + plsc.load_gather(tr_v, [lane17 + k])
            wA = wA_v[pl.ds(c * _C + g * 16, 16)]
            wB = wB_v[pl.ds(c * _C + g * 16, 16)]
            acc_total = acc_total + wA * d2 + wB * jnp.maximum(_MU - d2, 0.0)
        return acc_total

    start(0, 0)

    @pl.loop(0, nchunks // 2, init_carry=zero16)
    def chunk_loop(h, acc_total):
        for b in range(2):
            c = 2 * h + b

            @pl.when(c + 1 < nchunks)
            def _():
                start(c + 1, 1 - b)

            wait(b)
            acc_total = compute(c, b, acc_total)
        return acc_total

    acc_v[...] = chunk_loop
    pltpu.sync_copy(acc_v, out_hbm.at[wid])


def kernel(xS, xT, p_, n_):
    numP = p_.shape[0]
    numN = n_.shape[0]
    total = numP + numN
    assert total % (_NW * 2 * _C) == 0
    ppw = total // _NW

    yN = 0.2 * jax.random.uniform(jax.random.key(42), (numN,), dtype=jnp.float32)
    p32 = p_.astype(jnp.int32)
    n32 = n_.astype(jnp.int32)
    idx = jnp.concatenate([p32[:, 0], n32[:, 0], p32[:, 1], n32[:, 1]])
    w = jnp.concatenate([jnp.full((numP,), 1.0 / numP, jnp.float32),
                         yN / numN,
                         jnp.zeros((numP,), jnp.float32),
                         (1.0 - yN) / numN])

    mesh = plsc.VectorSubcoreMesh(core_axis_name="c", subcore_axis_name="s")
    run = pl.kernel(
        functools.partial(_tec_body, ppw=ppw, total=total),
        out_type=jax.ShapeDtypeStruct((_NW, 16), jnp.float32),
        mesh=mesh,
        compiler_params=pltpu.CompilerParams(use_tc_tiling_on_sc=False,
                                             needs_layout_passes=False),
        scratch_types=[
            pltpu.VMEM((ppw,), jnp.int32),
            pltpu.VMEM((ppw,), jnp.int32),
            pltpu.VMEM((ppw,), jnp.float32),
            pltpu.VMEM((ppw,), jnp.float32),
            pltpu.VMEM((2, _C, _D), jnp.bfloat16),
            pltpu.VMEM((2, _C, _D), jnp.bfloat16),
            pltpu.VMEM((16 * 17,), jnp.float32),
            pltpu.VMEM((16,), jnp.float32),
            pltpu.SemaphoreType.DMA((2,)),
            pltpu.SemaphoreType.DMA((2,)),
        ],
    )
    partials = run(xT.astype(jnp.bfloat16), xS.astype(jnp.bfloat16),
                   idx, w)
    return jnp.sum(partials, dtype=jnp.float32).reshape((1,))
